# baseline (speedup 1.0000x reference)
import functools

import jax
import jax.numpy as jnp
from jax import lax
from jax.experimental import pallas as pl
from jax.experimental.pallas import tpu as pltpu
from jax.experimental.pallas import tpu_sc as plsc

ROWS, COLS = 16384, 200
VOCAB, D = 25, 256
LANES = 16
RU = 2
B = ROWS * COLS
NC, NS = 2, 16
NW = NC * NS
B_PER_W = B // NW
CHUNK = 128
NCHUNK = B_PER_W // CHUNK
NBUF = 2
OUTER = NCHUNK // NBUF


@functools.partial(
    pl.kernel,
    out_type=jax.ShapeDtypeStruct((B, D), jnp.float32),
    mesh=plsc.VectorSubcoreMesh(
        core_axis_name="c", subcore_axis_name="s", num_cores=NC, num_subcores=NS
    ),
    scratch_types=[
        pltpu.VMEM((NBUF, CHUNK), jnp.int32),
        pltpu.VMEM((NBUF, CHUNK, D), jnp.float32),
        pltpu.VMEM((VOCAB * D,), jnp.float32),
        pltpu.SMEM((CHUNK,), jnp.int32),
    ]
    + [pltpu.SemaphoreType.DMA] * (2 * NBUF),
)
def _embed_expand(idx_hbm, table_hbm, out_hbm, idx_v, rows_v, table_f, soff,
                  *sems):
    sem_i = sems[0:NBUF]
    sem_s = sems[NBUF : 2 * NBUF]
    wid = lax.axis_index("s") * NC + lax.axis_index("c")
    base = wid * B_PER_W

    def idx_src(chunk):
        return idx_hbm.at[pl.ds(base + chunk * CHUNK, CHUNK)]

    def out_dst(chunk):
        return out_hbm.at[pl.ds(base + chunk * CHUNK, CHUNK), :]

    pltpu.sync_copy(table_hbm, table_f)

    for b in range(NBUF):
        pltpu.async_copy(idx_src(b), idx_v.at[b], sem_i[b])

    def outer(t, carry):
        for b in range(NBUF):
            i = t * NBUF + b

            @pl.when(t > 0)
            def _wait_prev_scatter():
                pltpu.make_async_copy(rows_v.at[b], out_dst(0), sem_s[b]).wait()

            pltpu.make_async_copy(idx_src(0), idx_v.at[b], sem_i[b]).wait()

            def stage(g, c2):
                ivec = idx_v[b, pl.ds(g * LANES, LANES)] * D
                for l in range(LANES):
                    soff[g * LANES + l] = ivec[l]
                return c2

            lax.fori_loop(0, CHUNK // LANES, stage, 0)

            def rows(q, c2):
                vals = []
                for u in range(RU):
                    s = soff[q * RU + u]
                    for c in range(D // LANES):
                        vals.append(table_f[pl.ds(s + c * LANES, LANES)])
                for u in range(RU):
                    r = q * RU + u
                    for c in range(D // LANES):
                        rows_v[b, r, pl.ds(c * LANES, LANES)] = vals[
                            u * (D // LANES) + c
                        ]
                return c2

            lax.fori_loop(0, CHUNK // RU, rows, 0)

            pltpu.async_copy(rows_v.at[b], out_dst(i), sem_s[b])
            nxt = jnp.minimum(i + NBUF, NCHUNK - 1)
            pltpu.async_copy(idx_src(nxt), idx_v.at[b], sem_i[b])
        return carry

    lax.fori_loop(0, OUTER, outer, 0)

    for b in range(NBUF):
        pltpu.make_async_copy(idx_src(0), idx_v.at[b], sem_i[b]).wait()
        pltpu.make_async_copy(rows_v.at[b], out_dst(0), sem_s[b]).wait()


def kernel(x, table):
    idx = x.reshape(B)
    out = _embed_expand(idx, table.reshape(VOCAB * D))
    return out.reshape(ROWS, COLS, D)
